# baseline (speedup 1.0000x reference)
import functools

import jax
import jax.numpy as jnp
from jax import lax
from jax.experimental import pallas as pl
from jax.experimental.pallas import tpu as pltpu
from jax.experimental.pallas import tpu_sc as plsc

B = 16
L = 200
EMBED = 200
HIDDEN = 256
DEC_VOCAB = 1000
VOCAB = 2000

_F32 = jnp.float32
_PREC = jax.lax.Precision.HIGHEST
_CONTRACT_T = (((1,), (1,)), ((), ()))
_LN2 = 0.6931471805599453


def _sigmoid(x):
    return 1.0 / (1.0 + jnp.exp(-x))


def _dotT(x, w, precision=_PREC):
    return lax.dot_general(x, w, _CONTRACT_T,
                           preferred_element_type=_F32, precision=precision)


def _dense_body(iid_ref, inp_ref, enc_ref, ids_ref, h0_ref, att0_ref,
                wih_ref, whh_ref, bih_ref, bhh_ref,
                attnw_ref, attnb_ref, combw_ref, combb_ref,
                genw_ref, genb_ref, copyw_ref, copyb_ref,
                gen_out_ref, copy_out_ref, ids_out_ref, hid_out_ref, att_out_ref):
    enc = enc_ref[...]
    ids = ids_ref[...]
    iid = iid_ref[...]

    mask = (iid == ids).astype(_F32)
    msum = jnp.sum(mask, axis=1, keepdims=True)
    rou = jnp.where(msum > 0, mask / jnp.where(msum > 0, msum, 1.0), 0.0)
    sel = jnp.sum(rou[:, :, None] * enc, axis=1)

    x = jnp.concatenate([inp_ref[:, 0, :], att0_ref[:, 0, :], sel], axis=1)
    gi = _dotT(x, wih_ref[...]) + bih_ref[...][None, :]
    h0 = h0_ref[:, 0, :]
    gh = _dotT(h0, whh_ref[...]) + bhh_ref[...][None, :]
    r = _sigmoid(gi[:, 0:HIDDEN] + gh[:, 0:HIDDEN])
    z = _sigmoid(gi[:, HIDDEN:2 * HIDDEN] + gh[:, HIDDEN:2 * HIDDEN])
    n = jnp.tanh(gi[:, 2 * HIDDEN:3 * HIDDEN] + r * gh[:, 2 * HIDDEN:3 * HIDDEN])
    hnew = (1.0 - z) * n + z * h0

    attn_q = _dotT(hnew, attnw_ref[...]) + attnb_ref[...][None, :]
    logits = jnp.sum(attn_q[:, None, :] * enc, axis=2)
    lmax = jnp.max(logits, axis=1, keepdims=True)
    le = jnp.exp(logits - lmax)
    aw = le / jnp.sum(le, axis=1, keepdims=True)
    attn_applied = jnp.sum(aw[:, :, None] * enc, axis=1)
    comb_in = jnp.concatenate([attn_applied, hnew], axis=1)
    cur_att = jnp.tanh(_dotT(comb_in, combw_ref[...]) + combb_ref[...][None, :])

    gen = _dotT(cur_att, genw_ref[...]) + genb_ref[...][None, :]
    cw = _sigmoid(_dotT(enc.reshape(B * L, HIDDEN), copyw_ref[...],
                        precision=jax.lax.Precision.DEFAULT)
                  + copyb_ref[...][None, :])
    cs = jnp.sum(cw.reshape(B, L, HIDDEN) * cur_att[:, None, :], axis=2)

    m = jnp.maximum(jnp.max(gen, axis=1, keepdims=True),
                    jnp.max(cs, axis=1, keepdims=True))
    eg = jnp.exp(gen - m)
    ec = jnp.exp(cs - m)
    denom = jnp.sum(eg, axis=1, keepdims=True) + jnp.sum(ec, axis=1, keepdims=True)
    gen_sm = eg / denom
    copy_sm = ec / denom

    for b in range(B):
        gen_out_ref[pl.ds(b * DEC_VOCAB, DEC_VOCAB)] = gen_sm[b:b + 1, :].reshape(DEC_VOCAB)
        copy_out_ref[pl.ds(b * L, L)] = copy_sm[b:b + 1, :].reshape(L)
        ids_out_ref[pl.ds(b * L, L)] = ids[b:b + 1, :].reshape(L)
    hid_out_ref[...] = hnew[:, None, :]
    att_out_ref[...] = cur_att[:, None, :]


def _sc_log(x):
    bits = plsc.bitcast(x, jnp.int32)
    e = lax.shift_right_arithmetic(bits, 23) - 127
    m = plsc.bitcast((bits & jnp.int32(0x007FFFFF)) | jnp.int32(0x3F800000), _F32)
    t = m - 1.0
    y = e.astype(_F32) * _LN2 + t * (1.0 - 0.5 * t)
    for _ in range(3):
        y = y + x * jnp.exp(-y) - 1.0
    return jnp.where(x > 0.0, y, -jnp.inf)


_NC = 2
_LANES = 16
_LP = 208
_NCHUNKS = _LP // _LANES


@functools.cache
def _make_sc_scatter():
    @functools.partial(
        pl.kernel,
        mesh=plsc.VectorSubcoreMesh(core_axis_name="c", subcore_axis_name="s"),
        out_type=jax.ShapeDtypeStruct((B * VOCAB,), jnp.float32),
        scratch_types=[
            pltpu.VMEM((_LP,), jnp.int32),
            pltpu.VMEM((_LP,), jnp.float32),
            pltpu.VMEM((VOCAB,), jnp.float32),
        ],
        compiler_params=pltpu.CompilerParams(needs_layout_passes=False),
    )
    def _sc_scatter(ids_hbm, cs_hbm, gen_hbm, out_hbm, ids_v, cs_v, row_v):
        wid = lax.axis_index("s") * _NC + lax.axis_index("c")

        @pl.when(wid < B)
        def _():
            zi = jnp.zeros((_LANES,), jnp.int32)
            zf = jnp.zeros((_LANES,), jnp.float32)
            ids_v[pl.ds(L - 8, _LANES)] = zi
            cs_v[pl.ds(L - 8, _LANES)] = zf
            pltpu.sync_copy(ids_hbm.at[pl.ds(wid * L, L)], ids_v.at[pl.ds(0, L)])
            pltpu.sync_copy(cs_hbm.at[pl.ds(wid * L, L)], cs_v.at[pl.ds(0, L)])

            def _zero(i, carry):
                row_v[pl.ds(i * _LANES, _LANES)] = zf
                return carry

            lax.fori_loop(0, VOCAB // _LANES, _zero, 0)
            pltpu.sync_copy(gen_hbm.at[pl.ds(wid * DEC_VOCAB, DEC_VOCAB)],
                            row_v.at[pl.ds(0, DEC_VOCAB)])

            for j in range(_NCHUNKS):
                idx = ids_v[pl.ds(j * _LANES, _LANES)]
                val = cs_v[pl.ds(j * _LANES, _LANES)]
                plsc.addupdate_scatter(row_v, [idx], val)

            def _logchunk(i, carry):
                row_v[pl.ds(i * _LANES, _LANES)] = _sc_log(row_v[pl.ds(i * _LANES, _LANES)])
                return carry

            lax.fori_loop(0, VOCAB // _LANES, _logchunk, 0)

            pltpu.sync_copy(row_v, out_hbm.at[pl.ds(wid * VOCAB, VOCAB)])

    return _sc_scatter


def kernel(input_id, input, encoder_outputs, encoder_input_ids, hidden, attention,
           W_ih, W_hh, b_ih, b_hh, attn_W, attn_b, comb_W, comb_b,
           gen_W, gen_b, copy_W, copy_b):
    gen_sm, copy_sm, ids_flat, hnew, cur_att = pl.pallas_call(
        _dense_body,
        out_shape=[
            jax.ShapeDtypeStruct((B * DEC_VOCAB,), jnp.float32),
            jax.ShapeDtypeStruct((B * L,), jnp.float32),
            jax.ShapeDtypeStruct((B * L,), jnp.int32),
            jax.ShapeDtypeStruct((B, 1, HIDDEN), jnp.float32),
            jax.ShapeDtypeStruct((B, 1, HIDDEN), jnp.float32),
        ],
    )(input_id.astype(jnp.int32), input, encoder_outputs,
      encoder_input_ids.astype(jnp.int32), hidden, attention,
      W_ih, W_hh, b_ih, b_hh, attn_W, attn_b, comb_W, comb_b,
      gen_W, gen_b, copy_W, copy_b)

    output = _make_sc_scatter()(ids_flat, copy_sm, gen_sm).reshape(B, VOCAB)

    return (output, hnew, cur_att)


# speedup vs baseline: 1.0509x; 1.0509x over previous
import functools

import jax
import jax.numpy as jnp
from jax import lax
from jax.experimental import pallas as pl
from jax.experimental.pallas import tpu as pltpu
from jax.experimental.pallas import tpu_sc as plsc

B = 16
L = 200
EMBED = 200
HIDDEN = 256
DEC_VOCAB = 1000
VOCAB = 2000

_F32 = jnp.float32
_BF16 = jnp.bfloat16
_CONTRACT_T = (((1,), (1,)), ((), ()))
_LN2 = 0.6931471805599453


def _sigmoid(x):
    return 1.0 / (1.0 + jnp.exp(-x))


def _bf(x):
    return x.astype(_BF16).astype(_F32)


def _dotT(x, w):
    return lax.dot_general(x.astype(_BF16), w.astype(_BF16), _CONTRACT_T,
                           preferred_element_type=_F32)


def _dense_body(iid_ref, inp_ref, enc_ref, ids_ref, h0_ref, att0_ref,
                wih_ref, whh_ref, bih_ref, bhh_ref,
                attnw_ref, attnb_ref, combw_ref, combb_ref,
                genw_ref, genb_ref, copyw_ref, copyb_ref,
                gen_out_ref, copy_out_ref, ids_out_ref, hid_out_ref, att_out_ref):
    enc = enc_ref[...]
    ids = ids_ref[...]
    iid = iid_ref[...]

    enc_bf = _bf(enc)

    mask = (iid == ids).astype(_F32)
    msum = jnp.sum(mask, axis=1, keepdims=True)
    rou = jnp.where(msum > 0, mask / jnp.where(msum > 0, msum, 1.0), 0.0)
    sel = jnp.sum(_bf(rou)[:, :, None] * enc_bf, axis=1)

    x = jnp.concatenate([inp_ref[:, 0, :], att0_ref[:, 0, :], sel], axis=1)
    gi = _dotT(x, wih_ref[...]) + bih_ref[...][None, :]
    h0 = h0_ref[:, 0, :]
    gh = _dotT(h0, whh_ref[...]) + bhh_ref[...][None, :]
    r = _sigmoid(gi[:, 0:HIDDEN] + gh[:, 0:HIDDEN])
    z = _sigmoid(gi[:, HIDDEN:2 * HIDDEN] + gh[:, HIDDEN:2 * HIDDEN])
    n = jnp.tanh(gi[:, 2 * HIDDEN:3 * HIDDEN] + r * gh[:, 2 * HIDDEN:3 * HIDDEN])
    hnew = (1.0 - z) * n + z * h0

    attn_q = _dotT(hnew, attnw_ref[...]) + attnb_ref[...][None, :]
    logits = jnp.sum(_bf(attn_q)[:, None, :] * enc_bf, axis=2)
    lmax = jnp.max(logits, axis=1, keepdims=True)
    le = jnp.exp(logits - lmax)
    aw = le / jnp.sum(le, axis=1, keepdims=True)
    attn_applied = jnp.sum(_bf(aw)[:, :, None] * enc_bf, axis=1)
    comb_in = jnp.concatenate([attn_applied, hnew], axis=1)
    cur_att = jnp.tanh(_dotT(comb_in, combw_ref[...]) + combb_ref[...][None, :])

    gen = _dotT(cur_att, genw_ref[...]) + genb_ref[...][None, :]
    cw = _sigmoid(_dotT(enc.reshape(B * L, HIDDEN), copyw_ref[...])
                  + copyb_ref[...][None, :])
    cs = jnp.sum(_bf(cw).reshape(B, L, HIDDEN) * _bf(cur_att)[:, None, :],
                 axis=2)

    m = jnp.maximum(jnp.max(gen, axis=1, keepdims=True),
                    jnp.max(cs, axis=1, keepdims=True))
    eg = jnp.exp(gen - m)
    ec = jnp.exp(cs - m)
    denom = jnp.sum(eg, axis=1, keepdims=True) + jnp.sum(ec, axis=1, keepdims=True)
    gen_sm = eg / denom
    copy_sm = ec / denom

    for b in range(B):
        gen_out_ref[pl.ds(b * DEC_VOCAB, DEC_VOCAB)] = gen_sm[b:b + 1, :].reshape(DEC_VOCAB)
        copy_out_ref[pl.ds(b * L, L)] = copy_sm[b:b + 1, :].reshape(L)
        ids_out_ref[pl.ds(b * L, L)] = ids[b:b + 1, :].reshape(L)
    hid_out_ref[...] = hnew[:, None, :]
    att_out_ref[...] = cur_att[:, None, :]


def _sc_log(x):
    bits = plsc.bitcast(x, jnp.int32)
    e = lax.shift_right_arithmetic(bits, 23) - 127
    m = plsc.bitcast((bits & jnp.int32(0x007FFFFF)) | jnp.int32(0x3F800000), _F32)
    t = m - 1.0
    y = e.astype(_F32) * _LN2 + t * (1.0 - 0.5 * t)
    for _ in range(3):
        y = y + x * jnp.exp(-y) - 1.0
    return jnp.where(x > 0.0, y, -jnp.inf)


_NC = 2
_LANES = 16
_LP = 208
_NCHUNKS = _LP // _LANES


@functools.cache
def _make_sc_scatter():
    @functools.partial(
        pl.kernel,
        mesh=plsc.VectorSubcoreMesh(core_axis_name="c", subcore_axis_name="s"),
        out_type=jax.ShapeDtypeStruct((B * VOCAB,), jnp.float32),
        scratch_types=[
            pltpu.VMEM((_LP,), jnp.int32),
            pltpu.VMEM((_LP,), jnp.float32),
            pltpu.VMEM((VOCAB,), jnp.float32),
        ],
        compiler_params=pltpu.CompilerParams(needs_layout_passes=False),
    )
    def _sc_scatter(ids_hbm, cs_hbm, gen_hbm, out_hbm, ids_v, cs_v, row_v):
        wid = lax.axis_index("s") * _NC + lax.axis_index("c")

        @pl.when(wid < B)
        def _():
            zi = jnp.zeros((_LANES,), jnp.int32)
            zf = jnp.zeros((_LANES,), jnp.float32)
            ids_v[pl.ds(L - 8, _LANES)] = zi
            cs_v[pl.ds(L - 8, _LANES)] = zf
            pltpu.sync_copy(ids_hbm.at[pl.ds(wid * L, L)], ids_v.at[pl.ds(0, L)])
            pltpu.sync_copy(cs_hbm.at[pl.ds(wid * L, L)], cs_v.at[pl.ds(0, L)])

            for i in range(VOCAB // _LANES):
                row_v[pl.ds(i * _LANES, _LANES)] = zf
            pltpu.sync_copy(gen_hbm.at[pl.ds(wid * DEC_VOCAB, DEC_VOCAB)],
                            row_v.at[pl.ds(0, DEC_VOCAB)])

            for j in range(_NCHUNKS):
                idx = ids_v[pl.ds(j * _LANES, _LANES)]
                val = cs_v[pl.ds(j * _LANES, _LANES)]
                plsc.addupdate_scatter(row_v, [idx], val)

            for i in range(VOCAB // _LANES):
                row_v[pl.ds(i * _LANES, _LANES)] = _sc_log(row_v[pl.ds(i * _LANES, _LANES)])

            pltpu.sync_copy(row_v, out_hbm.at[pl.ds(wid * VOCAB, VOCAB)])

    return _sc_scatter


def kernel(input_id, input, encoder_outputs, encoder_input_ids, hidden, attention,
           W_ih, W_hh, b_ih, b_hh, attn_W, attn_b, comb_W, comb_b,
           gen_W, gen_b, copy_W, copy_b):
    gen_sm, copy_sm, ids_flat, hnew, cur_att = pl.pallas_call(
        _dense_body,
        out_shape=[
            jax.ShapeDtypeStruct((B * DEC_VOCAB,), jnp.float32),
            jax.ShapeDtypeStruct((B * L,), jnp.float32),
            jax.ShapeDtypeStruct((B * L,), jnp.int32),
            jax.ShapeDtypeStruct((B, 1, HIDDEN), jnp.float32),
            jax.ShapeDtypeStruct((B, 1, HIDDEN), jnp.float32),
        ],
    )(input_id.astype(jnp.int32), input, encoder_outputs,
      encoder_input_ids.astype(jnp.int32), hidden, attention,
      W_ih, W_hh, b_ih, b_hh, attn_W, attn_b, comb_W, comb_b,
      gen_W, gen_b, copy_W, copy_b)

    output = _make_sc_scatter()(ids_flat, copy_sm, gen_sm).reshape(B, VOCAB)

    return (output, hnew, cur_att)


# speedup vs baseline: 1.1845x; 1.1271x over previous
import functools

import jax
import jax.numpy as jnp
from jax import lax
from jax.experimental import pallas as pl
from jax.experimental.pallas import tpu as pltpu
from jax.experimental.pallas import tpu_sc as plsc

B = 16
L = 200
EMBED = 200
HIDDEN = 256
DEC_VOCAB = 1000
VOCAB = 2000

_F32 = jnp.float32
_BF16 = jnp.bfloat16
_CONTRACT_T = (((1,), (1,)), ((), ()))
_LN2 = 0.6931471805599453


def _sigmoid(x):
    return 1.0 / (1.0 + jnp.exp(-x))


def _bf(x):
    return x.astype(_BF16).astype(_F32)


def _dotT(x, w):
    return lax.dot_general(x.astype(_BF16), w.astype(_BF16), _CONTRACT_T,
                           preferred_element_type=_F32)


def _dense_body(iid_ref, inp_ref, enc_ref, ids_ref, h0_ref, att0_ref,
                wih_ref, whh_ref, bih_ref, bhh_ref,
                attnw_ref, attnb_ref, combw_ref, combb_ref,
                genw_ref, genb_ref, copyw_ref, copyb_ref,
                gen_out_ref, copy_out_ref, ids_out_ref, hid_out_ref, att_out_ref):
    enc = enc_ref[...]
    ids = ids_ref[...]
    iid = iid_ref[...]

    enc_bf = _bf(enc)

    mask = (iid == ids).astype(_F32)
    msum = jnp.sum(mask, axis=1, keepdims=True)
    rou = jnp.where(msum > 0, mask / jnp.where(msum > 0, msum, 1.0), 0.0)
    sel = jnp.sum(_bf(rou)[:, :, None] * enc_bf, axis=1)

    x = jnp.concatenate([inp_ref[:, 0, :], att0_ref[:, 0, :], sel], axis=1)
    gi = _dotT(x, wih_ref[...]) + bih_ref[...][None, :]
    h0 = h0_ref[:, 0, :]
    gh = _dotT(h0, whh_ref[...]) + bhh_ref[...][None, :]
    r = _sigmoid(gi[:, 0:HIDDEN] + gh[:, 0:HIDDEN])
    z = _sigmoid(gi[:, HIDDEN:2 * HIDDEN] + gh[:, HIDDEN:2 * HIDDEN])
    n = jnp.tanh(gi[:, 2 * HIDDEN:3 * HIDDEN] + r * gh[:, 2 * HIDDEN:3 * HIDDEN])
    hnew = (1.0 - z) * n + z * h0

    attn_q = _dotT(hnew, attnw_ref[...]) + attnb_ref[...][None, :]
    logits = jnp.sum(_bf(attn_q)[:, None, :] * enc_bf, axis=2)
    lmax = jnp.max(logits, axis=1, keepdims=True)
    le = jnp.exp(logits - lmax)
    aw = le / jnp.sum(le, axis=1, keepdims=True)
    attn_applied = jnp.sum(_bf(aw)[:, :, None] * enc_bf, axis=1)
    comb_in = jnp.concatenate([attn_applied, hnew], axis=1)
    cur_att = jnp.tanh(_dotT(comb_in, combw_ref[...]) + combb_ref[...][None, :])

    gen = _dotT(cur_att, genw_ref[...]) + genb_ref[...][None, :]
    cw = _sigmoid(_dotT(enc.reshape(B * L, HIDDEN), copyw_ref[...])
                  + copyb_ref[...][None, :])
    cs = jnp.sum(_bf(cw).reshape(B, L, HIDDEN) * _bf(cur_att)[:, None, :],
                 axis=2)

    m = jnp.maximum(jnp.max(gen, axis=1, keepdims=True),
                    jnp.max(cs, axis=1, keepdims=True))
    eg = jnp.exp(gen - m)
    ec = jnp.exp(cs - m)
    denom = jnp.sum(eg, axis=1, keepdims=True) + jnp.sum(ec, axis=1, keepdims=True)
    gen_sm = eg / denom
    copy_sm = ec / denom

    for b in range(B):
        gen_out_ref[pl.ds(b * DEC_VOCAB, DEC_VOCAB)] = gen_sm[b:b + 1, :].reshape(DEC_VOCAB)
        copy_out_ref[pl.ds(b * L, L)] = copy_sm[b:b + 1, :].reshape(L)
        ids_out_ref[pl.ds(b * L, L)] = ids[b:b + 1, :].reshape(L)
    hid_out_ref[...] = hnew[:, None, :]
    att_out_ref[...] = cur_att[:, None, :]


def _sc_log(x):
    bits = plsc.bitcast(x, jnp.int32)
    e = lax.shift_right_arithmetic(bits, 23) - 127
    m = plsc.bitcast((bits & jnp.int32(0x007FFFFF)) | jnp.int32(0x3F800000), _F32)
    t = m - 1.0
    y = e.astype(_F32) * _LN2 + t * (1.0 - 0.5 * t)
    for _ in range(3):
        y = y + x * jnp.exp(-y) - 1.0
    return jnp.where(x > 0.0, y, -jnp.inf)


_NC = 2
_LANES = 16
_LP = 208
_NCHUNKS = _LP // _LANES


_HALF = VOCAB // 2
_HPAD = 1008


@functools.cache
def _make_sc_scatter():
    @functools.partial(
        pl.kernel,
        mesh=plsc.VectorSubcoreMesh(core_axis_name="c", subcore_axis_name="s"),
        out_type=jax.ShapeDtypeStruct((B * VOCAB,), jnp.float32),
        scratch_types=[
            pltpu.VMEM((_LP,), jnp.int32),
            pltpu.VMEM((_LP,), jnp.float32),
            pltpu.VMEM((_HPAD,), jnp.float32),
            pltpu.SemaphoreType.DMA,
        ],
        compiler_params=pltpu.CompilerParams(needs_layout_passes=False),
    )
    def _sc_scatter(ids_hbm, cs_hbm, gen_hbm, out_hbm, ids_v, cs_v, row_v, sem):
        wid = lax.axis_index("s") * _NC + lax.axis_index("c")
        b = lax.rem(wid, B)
        h = wid // B

        zi = jnp.zeros((_LANES,), jnp.int32)
        zf = jnp.zeros((_LANES,), jnp.float32)
        ids_v[pl.ds(L - 8, _LANES)] = zi
        cs_v[pl.ds(L - 8, _LANES)] = zf
        c1 = pltpu.async_copy(ids_hbm.at[pl.ds(b * L, L)],
                              ids_v.at[pl.ds(0, L)], sem)
        c2 = pltpu.async_copy(cs_hbm.at[pl.ds(b * L, L)],
                              cs_v.at[pl.ds(0, L)], sem)

        @pl.when(h == 0)
        def _():
            c3 = pltpu.async_copy(gen_hbm.at[pl.ds(b * DEC_VOCAB, DEC_VOCAB)],
                                  row_v.at[pl.ds(0, DEC_VOCAB)], sem)
            c3.wait()

        @pl.when(h == 1)
        def _():
            for i in range(_HPAD // _LANES):
                row_v[pl.ds(i * _LANES, _LANES)] = zf

        c1.wait()
        c2.wait()

        base = h * _HALF
        for j in range(_NCHUNKS):
            idx = ids_v[pl.ds(j * _LANES, _LANES)]
            val = cs_v[pl.ds(j * _LANES, _LANES)]
            in_half = (idx >= base) & (idx < base + _HALF)
            plsc.addupdate_scatter(row_v, [idx - base], val, mask=in_half)

        for i in range(_HPAD // _LANES):
            row_v[pl.ds(i * _LANES, _LANES)] = _sc_log(row_v[pl.ds(i * _LANES, _LANES)])

        pltpu.sync_copy(row_v.at[pl.ds(0, _HALF)],
                        out_hbm.at[pl.ds(b * VOCAB + base, _HALF)])

    return _sc_scatter


def kernel(input_id, input, encoder_outputs, encoder_input_ids, hidden, attention,
           W_ih, W_hh, b_ih, b_hh, attn_W, attn_b, comb_W, comb_b,
           gen_W, gen_b, copy_W, copy_b):
    gen_sm, copy_sm, ids_flat, hnew, cur_att = pl.pallas_call(
        _dense_body,
        out_shape=[
            jax.ShapeDtypeStruct((B * DEC_VOCAB,), jnp.float32),
            jax.ShapeDtypeStruct((B * L,), jnp.float32),
            jax.ShapeDtypeStruct((B * L,), jnp.int32),
            jax.ShapeDtypeStruct((B, 1, HIDDEN), jnp.float32),
            jax.ShapeDtypeStruct((B, 1, HIDDEN), jnp.float32),
        ],
    )(input_id, input, encoder_outputs, encoder_input_ids, hidden, attention,
      W_ih, W_hh, b_ih, b_hh, attn_W, attn_b, comb_W, comb_b,
      gen_W, gen_b, copy_W, copy_b)

    output = _make_sc_scatter()(ids_flat, copy_sm, gen_sm).reshape(B, VOCAB)

    return (output, hnew, cur_att)
